# Initial kernel scaffold; baseline (speedup 1.0000x reference)
import jax
import jax.numpy as jnp
from jax.experimental import pallas as pl


def kernel(length_q, length_k, embeddings_table):
    raise NotImplementedError("write your pallas kernel here")


# speedup vs baseline: 6.3993x; 6.3993x over previous
import jax
import jax.numpy as jnp
from jax import lax
from jax.experimental import pallas as pl
from jax.experimental.pallas import tpu as pltpu
from jax.experimental.pallas import tpu_sc as plsc

L_Q = 2048
L_K = 2048
D = 64
N_EMB = 129
SHIFT = L_K - 1 - (N_EMB - 1) // 2
N_SUB = 32
ROWS_PER_SUB = L_Q // N_SUB
HALF_K = L_K // 2
HALF_W = HALF_K * D
WIN_ROWS = HALF_K + ROWS_PER_SUB - 1
ROW_W = L_K * D


def _sc_body(table_hbm, out_hbm, table_v, win_v):
    c = lax.axis_index("c")
    s = lax.axis_index("s")
    wid = s * 2 + c
    base = wid * ROWS_PER_SUB

    pltpu.sync_copy(table_hbm, table_v)

    for half in range(2):
        win_lo = half * HALF_K + (L_K - 1 - (ROWS_PER_SUB - 1)) - base

        def build_row(w, carry):
            src = jnp.clip(win_lo + w - SHIFT, 0, N_EMB - 1) * D
            dst = w * D
            for q in range(D // 16):
                win_v[pl.ds(dst + q * 16, 16)] = table_v[pl.ds(src + q * 16, 16)]
            return carry

        lax.fori_loop(0, WIN_ROWS, build_row, 0)

        def copy_row(r, carry):
            src = pl.multiple_of((ROWS_PER_SUB - 1 - r) * D, D)
            dst = pl.multiple_of((base + r) * ROW_W + half * HALF_W, HALF_W)
            pltpu.sync_copy(
                win_v.at[pl.ds(src, HALF_W)],
                out_hbm.at[pl.ds(dst, HALF_W)],
            )
            return carry

        lax.fori_loop(0, ROWS_PER_SUB, copy_row, 0)


def kernel(length_q, length_k, embeddings_table):
    del length_q, length_k
    table_flat = embeddings_table.reshape(N_EMB * D)

    call = pl.kernel(
        _sc_body,
        out_type=jax.ShapeDtypeStruct((L_Q * ROW_W,), jnp.float32),
        mesh=plsc.VectorSubcoreMesh(core_axis_name="c", subcore_axis_name="s"),
        scratch_types=[
            pltpu.VMEM((N_EMB * D,), jnp.float32),
            pltpu.VMEM((WIN_ROWS * D,), jnp.float32),
        ],
    )
    out = call(table_flat)
    return out.reshape(L_Q, L_K, D)
